# Initial kernel scaffold; baseline (speedup 1.0000x reference)
import functools

import jax
import jax.numpy as jnp
from jax import lax
from jax.experimental import pallas as pl
from jax.experimental.pallas import tpu as pltpu
from jax.experimental.pallas import tpu_sc as plsc

N = 10000
E = 320000
D_IN = 128
D_HID = 128
D_OUT = 40

NC = 2
NS = 16
NW = NC * NS
E_PER_W = E // NW
K = 100
CHUNKS = E_PER_W // K
ROWS_PER_TILE = N // NS

_f32 = jnp.float32


def _sc_mesh():
    return plsc.VectorSubcoreMesh(core_axis_name="c", subcore_axis_name="s",
                                  num_cores=NC, num_subcores=NS)


@functools.partial(
    pl.kernel,
    out_type=[
        jax.ShapeDtypeStruct((NC, N, D_HID), _f32),
        jax.ShapeDtypeStruct((NC, N, 16), _f32),
    ],
    mesh=_sc_mesh(),
    scratch_types=[
        pltpu.VMEM((CHUNKS, K), jnp.int32),
        pltpu.VMEM((CHUNKS, K), jnp.int32),
        pltpu.VMEM((K, D_HID), _f32),
        pltpu.VMEM((K, 16), _f32),
        pltpu.VMEM_SHARED((N, D_HID), _f32),
        pltpu.VMEM_SHARED((N, 16), _f32),
        pltpu.SemaphoreType.DMA,
    ],
)
def _sc_agg1(h_hbm, src_hbm, dst_hbm, z_feat_hbm, z_deg_hbm, ones_hbm,
             out_hbm, deg_out_hbm,
             sidx, didx, rows, ones_v, acc, dacc, sem):
    c = lax.axis_index("c")
    s = lax.axis_index("s")
    w = c * NS + s

    pltpu.sync_copy(z_feat_hbm, acc.at[pl.ds(s * ROWS_PER_TILE, ROWS_PER_TILE)])
    pltpu.sync_copy(z_deg_hbm, dacc.at[pl.ds(s * ROWS_PER_TILE, ROWS_PER_TILE)])
    pltpu.sync_copy(src_hbm.at[w], sidx)
    pltpu.sync_copy(dst_hbm.at[w], didx)
    pltpu.sync_copy(ones_hbm, ones_v)
    plsc.subcore_barrier()

    def body(j, carry):
        pltpu.async_copy(h_hbm.at[sidx.at[j]], rows, sem).wait()
        pltpu.sync_copy(rows, acc.at[didx.at[j]], add=True)
        pltpu.sync_copy(ones_v, dacc.at[didx.at[j]], add=True)
        return carry

    lax.fori_loop(0, CHUNKS, body, 0)
    plsc.subcore_barrier()

    pltpu.sync_copy(acc.at[pl.ds(s * ROWS_PER_TILE, ROWS_PER_TILE)],
                    out_hbm.at[c, pl.ds(s * ROWS_PER_TILE, ROWS_PER_TILE)])
    pltpu.sync_copy(dacc.at[pl.ds(s * ROWS_PER_TILE, ROWS_PER_TILE)],
                    deg_out_hbm.at[c, pl.ds(s * ROWS_PER_TILE, ROWS_PER_TILE)])


@functools.partial(
    pl.kernel,
    out_type=jax.ShapeDtypeStruct((NC, N, D_OUT), _f32),
    mesh=_sc_mesh(),
    scratch_types=[
        pltpu.VMEM((CHUNKS, K), jnp.int32),
        pltpu.VMEM((CHUNKS, K), jnp.int32),
        pltpu.VMEM((K, D_OUT), _f32),
        pltpu.VMEM_SHARED((N, D_OUT), _f32),
        pltpu.SemaphoreType.DMA,
    ],
)
def _sc_agg2(h_hbm, src_hbm, dst_hbm, z_feat_hbm,
             out_hbm,
             sidx, didx, rows, acc, sem):
    c = lax.axis_index("c")
    s = lax.axis_index("s")
    w = c * NS + s

    pltpu.sync_copy(z_feat_hbm, acc.at[pl.ds(s * ROWS_PER_TILE, ROWS_PER_TILE)])
    pltpu.sync_copy(src_hbm.at[w], sidx)
    pltpu.sync_copy(dst_hbm.at[w], didx)
    plsc.subcore_barrier()

    def body(j, carry):
        pltpu.async_copy(h_hbm.at[sidx.at[j]], rows, sem).wait()
        pltpu.sync_copy(rows, acc.at[didx.at[j]], add=True)
        return carry

    lax.fori_loop(0, CHUNKS, body, 0)
    plsc.subcore_barrier()

    pltpu.sync_copy(acc.at[pl.ds(s * ROWS_PER_TILE, ROWS_PER_TILE)],
                    out_hbm.at[c, pl.ds(s * ROWS_PER_TILE, ROWS_PER_TILE)])


_BLK = 1000


def _mm1_body(x_ref, w_ref, o_ref):
    o_ref[...] = jnp.dot(x_ref[...], w_ref[...],
                         preferred_element_type=_f32)


def _tc_matmul1(x, w1):
    return pl.pallas_call(
        _mm1_body,
        grid=(N // _BLK,),
        in_specs=[
            pl.BlockSpec((_BLK, D_IN), lambda i: (i, 0)),
            pl.BlockSpec((D_IN, D_HID), lambda i: (0, 0)),
        ],
        out_specs=pl.BlockSpec((_BLK, D_HID), lambda i: (i, 0)),
        out_shape=jax.ShapeDtypeStruct((N, D_HID), _f32),
    )(x, w1)


def _mid_body(a0_ref, a1_ref, d0_ref, d1_ref, b1_ref, w2_ref, o_ref):
    deg = d0_ref[...] + d1_ref[...]
    deg_inv = 1.0 / jnp.maximum(deg, 1.0)
    x1 = (a0_ref[...] + a1_ref[...]) * deg_inv + b1_ref[...]
    x1 = jnp.maximum(x1, 0.0)
    o_ref[...] = jnp.dot(x1, w2_ref[...], preferred_element_type=_f32)


def _tc_mid(a0, a1, d0, d1, b1, w2):
    return pl.pallas_call(
        _mid_body,
        grid=(N // _BLK,),
        in_specs=[
            pl.BlockSpec((_BLK, D_HID), lambda i: (i, 0)),
            pl.BlockSpec((_BLK, D_HID), lambda i: (i, 0)),
            pl.BlockSpec((_BLK, 1), lambda i: (i, 0)),
            pl.BlockSpec((_BLK, 1), lambda i: (i, 0)),
            pl.BlockSpec((1, D_HID), lambda i: (0, 0)),
            pl.BlockSpec((D_HID, D_OUT), lambda i: (0, 0)),
        ],
        out_specs=pl.BlockSpec((_BLK, D_OUT), lambda i: (i, 0)),
        out_shape=jax.ShapeDtypeStruct((N, D_OUT), _f32),
    )(a0, a1, d0, d1, b1, w2)


def _final_body(g0_ref, g1_ref, d0_ref, d1_ref, b2_ref, o_ref):
    deg = d0_ref[...] + d1_ref[...]
    deg_inv = 1.0 / jnp.maximum(deg, 1.0)
    x = (g0_ref[...] + g1_ref[...]) * deg_inv + b2_ref[...]
    m = jnp.max(x, axis=1, keepdims=True)
    lse = m + jnp.log(jnp.sum(jnp.exp(x - m), axis=1, keepdims=True))
    o_ref[...] = x - lse


def _tc_final(g0, g1, d0, d1, b2):
    return pl.pallas_call(
        _final_body,
        grid=(N // _BLK,),
        in_specs=[
            pl.BlockSpec((_BLK, D_OUT), lambda i: (i, 0)),
            pl.BlockSpec((_BLK, D_OUT), lambda i: (i, 0)),
            pl.BlockSpec((_BLK, 1), lambda i: (i, 0)),
            pl.BlockSpec((_BLK, 1), lambda i: (i, 0)),
            pl.BlockSpec((1, D_OUT), lambda i: (0, 0)),
        ],
        out_specs=pl.BlockSpec((_BLK, D_OUT), lambda i: (i, 0)),
        out_shape=jax.ShapeDtypeStruct((N, D_OUT), _f32),
    )(g0, g1, d0, d1, b2)


def kernel(features, edge_index, W1, b1, W2, b2):
    src3 = edge_index[0].reshape(NW, CHUNKS, K)
    dst3 = edge_index[1].reshape(NW, CHUNKS, K)

    z_feat1 = jnp.zeros((ROWS_PER_TILE, D_HID), _f32)
    z_deg = jnp.zeros((ROWS_PER_TILE, 16), _f32)
    z_feat2 = jnp.zeros((ROWS_PER_TILE, D_OUT), _f32)
    ones_rows = jnp.ones((K, 16), _f32)

    h1 = _tc_matmul1(features, W1)
    agg1, deg16 = _sc_agg1(h1, src3, dst3, z_feat1, z_deg, ones_rows)

    d0 = deg16[0, :, 0:1]
    d1 = deg16[1, :, 0:1]
    h2 = _tc_mid(agg1[0], agg1[1], d0, d1, b1.reshape(1, D_HID), W2)

    agg2 = _sc_agg2(h2, src3, dst3, z_feat2)
    out = _tc_final(agg2[0], agg2[1], d0, d1, b2.reshape(1, D_OUT))
    return out


# speedup vs baseline: 6.4293x; 6.4293x over previous
import functools

import jax
import jax.numpy as jnp
from jax import lax
from jax.experimental import pallas as pl
from jax.experimental.pallas import tpu as pltpu
from jax.experimental.pallas import tpu_sc as plsc

N = 10000
E = 320000
D_IN = 128
D_HID = 128
D_HALF = D_HID // 2
D_OUT = 40

NC = 2
NS = 16
NW = NC * NS
K = 100
E_PER_TILE = E // NS
CH1 = E_PER_TILE // K
E_PER_W = E // NW
CH2 = E_PER_W // K
N_PAD = 10240
ROWS_PER_TILE = N_PAD // NS

_f32 = jnp.float32


def _sc_mesh():
    return plsc.VectorSubcoreMesh(core_axis_name="c", subcore_axis_name="s",
                                  num_cores=NC, num_subcores=NS)


@functools.partial(
    pl.kernel,
    out_type=[
        jax.ShapeDtypeStruct((NC, N_PAD, D_HALF), _f32),
        jax.ShapeDtypeStruct((NC, N_PAD, 16), _f32),
    ],
    mesh=_sc_mesh(),
    compiler_params=pltpu.CompilerParams(use_tc_tiling_on_sc=False),
    scratch_types=[
        pltpu.VMEM((CH1, K), jnp.int32),
        pltpu.VMEM((CH1, K), jnp.int32),
        pltpu.VMEM((K, D_HALF), _f32),
        pltpu.VMEM((K, 16), _f32),
        pltpu.VMEM_SHARED((N_PAD, D_HALF), _f32),
        pltpu.VMEM_SHARED((N_PAD, 16), _f32),
        pltpu.SemaphoreType.DMA,
    ],
)
def _sc_agg1(h_hbm, src_hbm, dst_hbm, z_feat_hbm, z_deg_hbm, ones_hbm,
             out_hbm, deg_out_hbm,
             sidx, didx, rows, ones_v, acc, dacc, sem):
    c = lax.axis_index("c")
    s = lax.axis_index("s")

    pltpu.sync_copy(z_feat_hbm, acc.at[pl.ds(s * ROWS_PER_TILE, ROWS_PER_TILE)])
    pltpu.sync_copy(z_deg_hbm, dacc.at[pl.ds(s * ROWS_PER_TILE, ROWS_PER_TILE)])
    pltpu.sync_copy(src_hbm.at[s], sidx)
    pltpu.sync_copy(dst_hbm.at[s], didx)
    pltpu.sync_copy(ones_hbm, ones_v)
    plsc.subcore_barrier()

    def body(j, carry):
        pltpu.async_copy(h_hbm.at[c].at[sidx.at[j]], rows, sem).wait()
        pltpu.sync_copy(rows, acc.at[didx.at[j]], add=True)
        pltpu.sync_copy(ones_v, dacc.at[didx.at[j]], add=True)
        return carry

    lax.fori_loop(0, CH1, body, 0)
    plsc.subcore_barrier()

    pltpu.sync_copy(acc.at[pl.ds(s * ROWS_PER_TILE, ROWS_PER_TILE)],
                    out_hbm.at[c, pl.ds(s * ROWS_PER_TILE, ROWS_PER_TILE)])
    pltpu.sync_copy(dacc.at[pl.ds(s * ROWS_PER_TILE, ROWS_PER_TILE)],
                    deg_out_hbm.at[c, pl.ds(s * ROWS_PER_TILE, ROWS_PER_TILE)])


@functools.partial(
    pl.kernel,
    out_type=jax.ShapeDtypeStruct((NC, N_PAD, D_OUT), _f32),
    mesh=_sc_mesh(),
    compiler_params=pltpu.CompilerParams(use_tc_tiling_on_sc=False),
    scratch_types=[
        pltpu.VMEM((CH2, K), jnp.int32),
        pltpu.VMEM((CH2, K), jnp.int32),
        pltpu.VMEM((K, D_OUT), _f32),
        pltpu.VMEM_SHARED((N_PAD, D_OUT), _f32),
        pltpu.SemaphoreType.DMA,
    ],
)
def _sc_agg2(h_hbm, src_hbm, dst_hbm, z_feat_hbm,
             out_hbm,
             sidx, didx, rows, acc, sem):
    c = lax.axis_index("c")
    s = lax.axis_index("s")
    w = c * NS + s

    pltpu.sync_copy(z_feat_hbm, acc.at[pl.ds(s * ROWS_PER_TILE, ROWS_PER_TILE)])
    pltpu.sync_copy(src_hbm.at[w], sidx)
    pltpu.sync_copy(dst_hbm.at[w], didx)
    plsc.subcore_barrier()

    def body(j, carry):
        pltpu.async_copy(h_hbm.at[sidx.at[j]], rows, sem).wait()
        pltpu.sync_copy(rows, acc.at[didx.at[j]], add=True)
        return carry

    lax.fori_loop(0, CH2, body, 0)
    plsc.subcore_barrier()

    pltpu.sync_copy(acc.at[pl.ds(s * ROWS_PER_TILE, ROWS_PER_TILE)],
                    out_hbm.at[c, pl.ds(s * ROWS_PER_TILE, ROWS_PER_TILE)])


_BLK = 1000


def _mm1_body(x_ref, w_ref, o_ref):
    h = jnp.dot(x_ref[...], w_ref[...], preferred_element_type=_f32)
    o_ref[0] = h[:, :D_HALF]
    o_ref[1] = h[:, D_HALF:]


def _tc_matmul1(x, w1):
    return pl.pallas_call(
        _mm1_body,
        grid=(N // _BLK,),
        in_specs=[
            pl.BlockSpec((_BLK, D_IN), lambda i: (i, 0)),
            pl.BlockSpec((D_IN, D_HID), lambda i: (0, 0)),
        ],
        out_specs=pl.BlockSpec((2, _BLK, D_HALF), lambda i: (0, i, 0)),
        out_shape=jax.ShapeDtypeStruct((2, N, D_HALF), _f32),
    )(x, w1)


def _mid_body(a0_ref, a1_ref, d_ref, b1_ref, w2_ref, o_ref):
    deg_inv = 1.0 / jnp.maximum(d_ref[...], 1.0)
    x1 = jnp.concatenate([a0_ref[...], a1_ref[...]], axis=1) * deg_inv \
        + b1_ref[...]
    x1 = jnp.maximum(x1, 0.0)
    o_ref[...] = jnp.dot(x1, w2_ref[...], preferred_element_type=_f32)


def _tc_mid(a0, a1, d, b1, w2):
    return pl.pallas_call(
        _mid_body,
        grid=(N // _BLK,),
        in_specs=[
            pl.BlockSpec((_BLK, D_HALF), lambda i: (i, 0)),
            pl.BlockSpec((_BLK, D_HALF), lambda i: (i, 0)),
            pl.BlockSpec((_BLK, 1), lambda i: (i, 0)),
            pl.BlockSpec((1, D_HID), lambda i: (0, 0)),
            pl.BlockSpec((D_HID, D_OUT), lambda i: (0, 0)),
        ],
        out_specs=pl.BlockSpec((_BLK, D_OUT), lambda i: (i, 0)),
        out_shape=jax.ShapeDtypeStruct((N, D_OUT), _f32),
    )(a0, a1, d, b1, w2)


def _final_body(g0_ref, g1_ref, d_ref, b2_ref, o_ref):
    deg_inv = 1.0 / jnp.maximum(d_ref[...], 1.0)
    x = (g0_ref[...] + g1_ref[...]) * deg_inv + b2_ref[...]
    m = jnp.max(x, axis=1, keepdims=True)
    lse = m + jnp.log(jnp.sum(jnp.exp(x - m), axis=1, keepdims=True))
    o_ref[...] = x - lse


def _tc_final(g0, g1, d, b2):
    return pl.pallas_call(
        _final_body,
        grid=(N // _BLK,),
        in_specs=[
            pl.BlockSpec((_BLK, D_OUT), lambda i: (i, 0)),
            pl.BlockSpec((_BLK, D_OUT), lambda i: (i, 0)),
            pl.BlockSpec((_BLK, 1), lambda i: (i, 0)),
            pl.BlockSpec((1, D_OUT), lambda i: (0, 0)),
        ],
        out_specs=pl.BlockSpec((_BLK, D_OUT), lambda i: (i, 0)),
        out_shape=jax.ShapeDtypeStruct((N, D_OUT), _f32),
    )(g0, g1, d, b2)


def kernel(features, edge_index, W1, b1, W2, b2):
    src_t = edge_index[0].reshape(NS, CH1, K)
    dst_t = edge_index[1].reshape(NS, CH1, K)
    src_w = edge_index[0].reshape(NW, CH2, K)
    dst_w = edge_index[1].reshape(NW, CH2, K)

    z_half = jnp.zeros((ROWS_PER_TILE, D_HALF), _f32)
    z_deg = jnp.zeros((ROWS_PER_TILE, 16), _f32)
    z_out = jnp.zeros((ROWS_PER_TILE, D_OUT), _f32)
    ones_rows = jnp.ones((K, 16), _f32)

    h1 = _tc_matmul1(features, W1)
    agg1, deg16 = _sc_agg1(h1, src_t, dst_t, z_half, z_deg, ones_rows)

    d = deg16[0, :N, 0:1]
    h2 = _tc_mid(agg1[0, :N], agg1[1, :N], d, b1.reshape(1, D_HID), W2)

    agg2 = _sc_agg2(h2, src_w, dst_w, z_out)
    out = _tc_final(agg2[0, :N], agg2[1, :N], d, b2.reshape(1, D_OUT))
    return out


# speedup vs baseline: 9.6138x; 1.4953x over previous
import functools

import jax
import jax.numpy as jnp
from jax import lax
from jax.experimental import pallas as pl
from jax.experimental.pallas import tpu as pltpu
from jax.experimental.pallas import tpu_sc as plsc

N = 10000
E = 320000
D_IN = 128
D_HID = 128
D_HALF = D_HID // 2
D_OUT = 40

NC = 2
NS = 16
NW = NC * NS
K = 100
E_PER_TILE = E // NS
CH1 = E_PER_TILE // K
E_PER_W = E // NW
CH2 = E_PER_W // K
N_PAD = 10240
ROWS_PER_TILE = N_PAD // NS

_f32 = jnp.float32


def _sc_mesh():
    return plsc.VectorSubcoreMesh(core_axis_name="c", subcore_axis_name="s",
                                  num_cores=NC, num_subcores=NS)


@functools.partial(
    pl.kernel,
    out_type=[
        jax.ShapeDtypeStruct((NC, N_PAD, D_HALF), _f32),
        jax.ShapeDtypeStruct((NC, N_PAD, 16), _f32),
    ],
    mesh=_sc_mesh(),
    compiler_params=pltpu.CompilerParams(use_tc_tiling_on_sc=False),
    scratch_types=[
        pltpu.VMEM((CH1, K), jnp.int32),
        pltpu.VMEM((CH1, K), jnp.int32),
        pltpu.VMEM((K, D_HALF), _f32),
        pltpu.VMEM((K, D_HALF), _f32),
        pltpu.VMEM((K, 16), _f32),
        pltpu.VMEM_SHARED((N_PAD, D_HALF), _f32),
        pltpu.VMEM_SHARED((N_PAD, 16), _f32),
        pltpu.SemaphoreType.DMA,
        pltpu.SemaphoreType.DMA,
        pltpu.SemaphoreType.DMA,
        pltpu.SemaphoreType.DMA,
    ],
)
def _sc_agg1(h_hbm, src_hbm, dst_hbm, z_feat_hbm, z_deg_hbm, ones_hbm,
             out_hbm, deg_out_hbm,
             sidx, didx, rows0, rows1, ones_v, acc, dacc,
             g0, g1, s0, s1):
    c = lax.axis_index("c")
    s = lax.axis_index("s")

    pltpu.sync_copy(z_feat_hbm, acc.at[pl.ds(s * ROWS_PER_TILE, ROWS_PER_TILE)])
    pltpu.sync_copy(z_deg_hbm, dacc.at[pl.ds(s * ROWS_PER_TILE, ROWS_PER_TILE)])
    pltpu.sync_copy(src_hbm.at[s], sidx)
    pltpu.sync_copy(dst_hbm.at[s], didx)
    pltpu.sync_copy(ones_hbm, ones_v)
    plsc.subcore_barrier()

    pltpu.async_copy(h_hbm.at[c].at[sidx.at[0]], rows0, g0)
    pltpu.async_copy(h_hbm.at[c].at[sidx.at[1]], rows1, g1)

    def halfstep(j, rows, gsem, ssem):
        pltpu.make_async_copy(h_hbm.at[c].at[sidx.at[j]], rows, gsem).wait()
        pltpu.async_copy(rows, acc.at[didx.at[j]], ssem, add=True)
        pltpu.async_copy(ones_v, dacc.at[didx.at[j]], ssem, add=True)
        pltpu.make_async_copy(rows, acc.at[didx.at[j]], ssem).wait()
        pltpu.make_async_copy(ones_v, dacc.at[didx.at[j]], ssem).wait()
        nxt = jnp.minimum(j + 2, CH1 - 1)
        pltpu.async_copy(h_hbm.at[c].at[sidx.at[nxt]], rows, gsem)

    def body(t, carry):
        halfstep(2 * t, rows0, g0, s0)
        halfstep(2 * t + 1, rows1, g1, s1)
        return carry

    lax.fori_loop(0, CH1 // 2, body, 0)
    pltpu.make_async_copy(h_hbm.at[c].at[sidx.at[0]], rows0, g0).wait()
    pltpu.make_async_copy(h_hbm.at[c].at[sidx.at[0]], rows1, g1).wait()
    plsc.subcore_barrier()

    pltpu.sync_copy(acc.at[pl.ds(s * ROWS_PER_TILE, ROWS_PER_TILE)],
                    out_hbm.at[c, pl.ds(s * ROWS_PER_TILE, ROWS_PER_TILE)])
    pltpu.sync_copy(dacc.at[pl.ds(s * ROWS_PER_TILE, ROWS_PER_TILE)],
                    deg_out_hbm.at[c, pl.ds(s * ROWS_PER_TILE, ROWS_PER_TILE)])


@functools.partial(
    pl.kernel,
    out_type=jax.ShapeDtypeStruct((NC, N_PAD, D_OUT), _f32),
    mesh=_sc_mesh(),
    compiler_params=pltpu.CompilerParams(use_tc_tiling_on_sc=False),
    scratch_types=[
        pltpu.VMEM((CH2, K), jnp.int32),
        pltpu.VMEM((CH2, K), jnp.int32),
        pltpu.VMEM((K, D_OUT), _f32),
        pltpu.VMEM((K, D_OUT), _f32),
        pltpu.VMEM_SHARED((N_PAD, D_OUT), _f32),
        pltpu.SemaphoreType.DMA,
        pltpu.SemaphoreType.DMA,
        pltpu.SemaphoreType.DMA,
        pltpu.SemaphoreType.DMA,
    ],
)
def _sc_agg2(h_hbm, src_hbm, dst_hbm, z_feat_hbm,
             out_hbm,
             sidx, didx, rows0, rows1, acc, g0, g1, s0, s1):
    c = lax.axis_index("c")
    s = lax.axis_index("s")
    w = c * NS + s

    pltpu.sync_copy(z_feat_hbm, acc.at[pl.ds(s * ROWS_PER_TILE, ROWS_PER_TILE)])
    pltpu.sync_copy(src_hbm.at[w], sidx)
    pltpu.sync_copy(dst_hbm.at[w], didx)
    plsc.subcore_barrier()

    pltpu.async_copy(h_hbm.at[sidx.at[0]], rows0, g0)
    pltpu.async_copy(h_hbm.at[sidx.at[1]], rows1, g1)

    def halfstep(j, rows, gsem, ssem):
        pltpu.make_async_copy(h_hbm.at[sidx.at[j]], rows, gsem).wait()
        pltpu.async_copy(rows, acc.at[didx.at[j]], ssem, add=True)
        pltpu.make_async_copy(rows, acc.at[didx.at[j]], ssem).wait()
        nxt = jnp.minimum(j + 2, CH2 - 1)
        pltpu.async_copy(h_hbm.at[sidx.at[nxt]], rows, gsem)

    def body(t, carry):
        halfstep(2 * t, rows0, g0, s0)
        halfstep(2 * t + 1, rows1, g1, s1)
        return carry

    lax.fori_loop(0, CH2 // 2, body, 0)
    pltpu.make_async_copy(h_hbm.at[sidx.at[0]], rows0, g0).wait()
    pltpu.make_async_copy(h_hbm.at[sidx.at[0]], rows1, g1).wait()
    plsc.subcore_barrier()

    pltpu.sync_copy(acc.at[pl.ds(s * ROWS_PER_TILE, ROWS_PER_TILE)],
                    out_hbm.at[c, pl.ds(s * ROWS_PER_TILE, ROWS_PER_TILE)])


_BLK = 1000


def _mm1_body(x_ref, w_ref, o_ref):
    h = jnp.dot(x_ref[...], w_ref[...], preferred_element_type=_f32)
    o_ref[0] = h[:, :D_HALF]
    o_ref[1] = h[:, D_HALF:]


def _tc_matmul1(x, w1):
    return pl.pallas_call(
        _mm1_body,
        grid=(N // _BLK,),
        in_specs=[
            pl.BlockSpec((_BLK, D_IN), lambda i: (i, 0)),
            pl.BlockSpec((D_IN, D_HID), lambda i: (0, 0)),
        ],
        out_specs=pl.BlockSpec((2, _BLK, D_HALF), lambda i: (0, i, 0)),
        out_shape=jax.ShapeDtypeStruct((2, N, D_HALF), _f32),
    )(x, w1)


def _mid_body(a0_ref, a1_ref, d_ref, b1_ref, w2_ref, o_ref):
    deg_inv = 1.0 / jnp.maximum(d_ref[...], 1.0)
    x1 = jnp.concatenate([a0_ref[...], a1_ref[...]], axis=1) * deg_inv \
        + b1_ref[...]
    x1 = jnp.maximum(x1, 0.0)
    o_ref[...] = jnp.dot(x1, w2_ref[...], preferred_element_type=_f32)


def _tc_mid(a0, a1, d, b1, w2):
    return pl.pallas_call(
        _mid_body,
        grid=(N // _BLK,),
        in_specs=[
            pl.BlockSpec((_BLK, D_HALF), lambda i: (i, 0)),
            pl.BlockSpec((_BLK, D_HALF), lambda i: (i, 0)),
            pl.BlockSpec((_BLK, 1), lambda i: (i, 0)),
            pl.BlockSpec((1, D_HID), lambda i: (0, 0)),
            pl.BlockSpec((D_HID, D_OUT), lambda i: (0, 0)),
        ],
        out_specs=pl.BlockSpec((_BLK, D_OUT), lambda i: (i, 0)),
        out_shape=jax.ShapeDtypeStruct((N, D_OUT), _f32),
    )(a0, a1, d, b1, w2)


def _final_body(g0_ref, g1_ref, d_ref, b2_ref, o_ref):
    deg_inv = 1.0 / jnp.maximum(d_ref[...], 1.0)
    x = (g0_ref[...] + g1_ref[...]) * deg_inv + b2_ref[...]
    m = jnp.max(x, axis=1, keepdims=True)
    lse = m + jnp.log(jnp.sum(jnp.exp(x - m), axis=1, keepdims=True))
    o_ref[...] = x - lse


def _tc_final(g0, g1, d, b2):
    return pl.pallas_call(
        _final_body,
        grid=(N // _BLK,),
        in_specs=[
            pl.BlockSpec((_BLK, D_OUT), lambda i: (i, 0)),
            pl.BlockSpec((_BLK, D_OUT), lambda i: (i, 0)),
            pl.BlockSpec((_BLK, 1), lambda i: (i, 0)),
            pl.BlockSpec((1, D_OUT), lambda i: (0, 0)),
        ],
        out_specs=pl.BlockSpec((_BLK, D_OUT), lambda i: (i, 0)),
        out_shape=jax.ShapeDtypeStruct((N, D_OUT), _f32),
    )(g0, g1, d, b2)


def kernel(features, edge_index, W1, b1, W2, b2):
    src_t = edge_index[0].reshape(NS, CH1, K)
    dst_t = edge_index[1].reshape(NS, CH1, K)
    src_w = edge_index[0].reshape(NW, CH2, K)
    dst_w = edge_index[1].reshape(NW, CH2, K)

    z_half = jnp.zeros((ROWS_PER_TILE, D_HALF), _f32)
    z_deg = jnp.zeros((ROWS_PER_TILE, 16), _f32)
    z_out = jnp.zeros((ROWS_PER_TILE, D_OUT), _f32)
    ones_rows = jnp.ones((K, 16), _f32)

    h1 = _tc_matmul1(features, W1)
    agg1, deg16 = _sc_agg1(h1, src_t, dst_t, z_half, z_deg, ones_rows)

    d = deg16[0, :N, 0:1]
    h2 = _tc_mid(agg1[0, :N], agg1[1, :N], d, b1.reshape(1, D_HID), W2)

    agg2 = _sc_agg2(h2, src_w, dst_w, z_out)
    out = _tc_final(agg2[0, :N], agg2[1, :N], d, b2.reshape(1, D_OUT))
    return out


# speedup vs baseline: 11.2202x; 1.1671x over previous
import functools

import jax
import jax.numpy as jnp
from jax import lax
from jax.experimental import pallas as pl
from jax.experimental.pallas import tpu as pltpu
from jax.experimental.pallas import tpu_sc as plsc

N = 10000
E = 320000
D_IN = 128
D_HID = 128
D_HALF = D_HID // 2
D_OUT = 40

NC = 2
NS = 16
NW = NC * NS
K = 100
E_PER_TILE = E // NS
CH1 = E_PER_TILE // K
E_PER_W = E // NW
CH2 = E_PER_W // K
N_PAD = 10240
ROWS_PER_TILE = N_PAD // NS
NBUF = 4

_f32 = jnp.float32


def _sc_mesh():
    return plsc.VectorSubcoreMesh(core_axis_name="c", subcore_axis_name="s",
                                  num_cores=NC, num_subcores=NS)


def _pipelined_agg(ch, gather_start, gather_wait, scat_start, scat_wait):
    for u in range(NBUF - 1):
        gather_start(u, u)

    def step(j, b):
        gather_wait(j, b)
        scat_start(j, b)
        if not (isinstance(j, int) and j == 0):
            scat_wait(None, (b - 1) % NBUF)
        gather_start(jnp.minimum(j + NBUF - 1, ch - 1), (b + NBUF - 1) % NBUF)

    for j in range(NBUF):
        step(j, j % NBUF)

    def body(t, carry):
        for u in range(NBUF):
            step(t * NBUF + u, u)
        return carry

    lax.fori_loop(1, ch // NBUF, body, 0)

    scat_wait(None, (ch - 1) % NBUF)
    for u in range(NBUF - 1):
        gather_wait(0, u)


@functools.partial(
    pl.kernel,
    out_type=[
        jax.ShapeDtypeStruct((NC, N_PAD, D_HALF), _f32),
        jax.ShapeDtypeStruct((NC, N_PAD, 16), _f32),
    ],
    mesh=_sc_mesh(),
    compiler_params=pltpu.CompilerParams(use_tc_tiling_on_sc=False),
    scratch_types=[
        pltpu.VMEM((CH1, K), jnp.int32),
        pltpu.VMEM((CH1, K), jnp.int32),
        [pltpu.VMEM((K, D_HALF), _f32)] * NBUF,
        pltpu.VMEM((K, 16), _f32),
        pltpu.VMEM_SHARED((N_PAD, D_HALF), _f32),
        pltpu.VMEM_SHARED((N_PAD, 16), _f32),
        [pltpu.SemaphoreType.DMA] * NBUF,
        [pltpu.SemaphoreType.DMA] * NBUF,
    ],
)
def _sc_agg1(h_hbm, src_hbm, dst_hbm, z_feat_hbm, z_deg_hbm, ones_hbm,
             out_hbm, deg_out_hbm,
             sidx, didx, rows, ones_v, acc, dacc, gsem, ssem):
    c = lax.axis_index("c")
    s = lax.axis_index("s")

    pltpu.sync_copy(z_feat_hbm, acc.at[pl.ds(s * ROWS_PER_TILE, ROWS_PER_TILE)])
    pltpu.sync_copy(z_deg_hbm, dacc.at[pl.ds(s * ROWS_PER_TILE, ROWS_PER_TILE)])
    pltpu.sync_copy(src_hbm.at[s], sidx)
    pltpu.sync_copy(dst_hbm.at[s], didx)
    pltpu.sync_copy(ones_hbm, ones_v)
    plsc.subcore_barrier()

    def gather_start(j, b):
        pltpu.async_copy(h_hbm.at[c].at[sidx.at[j]], rows[b], gsem[b])

    def gather_wait(j, b):
        pltpu.make_async_copy(h_hbm.at[c].at[sidx.at[0]], rows[b],
                              gsem[b]).wait()

    def scat_start(j, b):
        pltpu.async_copy(rows[b], acc.at[didx.at[j]], ssem[b], add=True)
        pltpu.async_copy(ones_v, dacc.at[didx.at[j]], ssem[b], add=True)

    def scat_wait(_, b):
        pltpu.make_async_copy(rows[b], acc.at[didx.at[0]], ssem[b]).wait()
        pltpu.make_async_copy(ones_v, dacc.at[didx.at[0]], ssem[b]).wait()

    _pipelined_agg(CH1, gather_start, gather_wait, scat_start, scat_wait)
    plsc.subcore_barrier()

    pltpu.sync_copy(acc.at[pl.ds(s * ROWS_PER_TILE, ROWS_PER_TILE)],
                    out_hbm.at[c, pl.ds(s * ROWS_PER_TILE, ROWS_PER_TILE)])
    pltpu.sync_copy(dacc.at[pl.ds(s * ROWS_PER_TILE, ROWS_PER_TILE)],
                    deg_out_hbm.at[c, pl.ds(s * ROWS_PER_TILE, ROWS_PER_TILE)])


@functools.partial(
    pl.kernel,
    out_type=jax.ShapeDtypeStruct((NC, N_PAD, D_OUT), _f32),
    mesh=_sc_mesh(),
    compiler_params=pltpu.CompilerParams(use_tc_tiling_on_sc=False),
    scratch_types=[
        pltpu.VMEM((CH2, K), jnp.int32),
        pltpu.VMEM((CH2, K), jnp.int32),
        [pltpu.VMEM((K, D_OUT), _f32)] * NBUF,
        pltpu.VMEM_SHARED((N_PAD, D_OUT), _f32),
        [pltpu.SemaphoreType.DMA] * NBUF,
        [pltpu.SemaphoreType.DMA] * NBUF,
    ],
)
def _sc_agg2(h_hbm, src_hbm, dst_hbm, z_feat_hbm,
             out_hbm,
             sidx, didx, rows, acc, gsem, ssem):
    c = lax.axis_index("c")
    s = lax.axis_index("s")
    w = c * NS + s

    pltpu.sync_copy(z_feat_hbm, acc.at[pl.ds(s * ROWS_PER_TILE, ROWS_PER_TILE)])
    pltpu.sync_copy(src_hbm.at[w], sidx)
    pltpu.sync_copy(dst_hbm.at[w], didx)
    plsc.subcore_barrier()

    def gather_start(j, b):
        pltpu.async_copy(h_hbm.at[sidx.at[j]], rows[b], gsem[b])

    def gather_wait(j, b):
        pltpu.make_async_copy(h_hbm.at[sidx.at[0]], rows[b], gsem[b]).wait()

    def scat_start(j, b):
        pltpu.async_copy(rows[b], acc.at[didx.at[j]], ssem[b], add=True)

    def scat_wait(_, b):
        pltpu.make_async_copy(rows[b], acc.at[didx.at[0]], ssem[b]).wait()

    _pipelined_agg(CH2, gather_start, gather_wait, scat_start, scat_wait)
    plsc.subcore_barrier()

    pltpu.sync_copy(acc.at[pl.ds(s * ROWS_PER_TILE, ROWS_PER_TILE)],
                    out_hbm.at[c, pl.ds(s * ROWS_PER_TILE, ROWS_PER_TILE)])


_BLK = 1000


def _mm1_body(x_ref, w_ref, o_ref):
    h = jnp.dot(x_ref[...], w_ref[...], preferred_element_type=_f32)
    o_ref[0] = h[:, :D_HALF]
    o_ref[1] = h[:, D_HALF:]


def _tc_matmul1(x, w1):
    return pl.pallas_call(
        _mm1_body,
        grid=(N // _BLK,),
        in_specs=[
            pl.BlockSpec((_BLK, D_IN), lambda i: (i, 0)),
            pl.BlockSpec((D_IN, D_HID), lambda i: (0, 0)),
        ],
        out_specs=pl.BlockSpec((2, _BLK, D_HALF), lambda i: (0, i, 0)),
        out_shape=jax.ShapeDtypeStruct((2, N, D_HALF), _f32),
    )(x, w1)


def _mid_body(a0_ref, a1_ref, d_ref, b1_ref, w2_ref, o_ref):
    deg_inv = 1.0 / jnp.maximum(d_ref[...], 1.0)
    x1 = jnp.concatenate([a0_ref[...], a1_ref[...]], axis=1) * deg_inv \
        + b1_ref[...]
    x1 = jnp.maximum(x1, 0.0)
    o_ref[...] = jnp.dot(x1, w2_ref[...], preferred_element_type=_f32)


def _tc_mid(a0, a1, d, b1, w2):
    return pl.pallas_call(
        _mid_body,
        grid=(N // _BLK,),
        in_specs=[
            pl.BlockSpec((_BLK, D_HALF), lambda i: (i, 0)),
            pl.BlockSpec((_BLK, D_HALF), lambda i: (i, 0)),
            pl.BlockSpec((_BLK, 1), lambda i: (i, 0)),
            pl.BlockSpec((1, D_HID), lambda i: (0, 0)),
            pl.BlockSpec((D_HID, D_OUT), lambda i: (0, 0)),
        ],
        out_specs=pl.BlockSpec((_BLK, D_OUT), lambda i: (i, 0)),
        out_shape=jax.ShapeDtypeStruct((N, D_OUT), _f32),
    )(a0, a1, d, b1, w2)


def _final_body(g0_ref, g1_ref, d_ref, b2_ref, o_ref):
    deg_inv = 1.0 / jnp.maximum(d_ref[...], 1.0)
    x = (g0_ref[...] + g1_ref[...]) * deg_inv + b2_ref[...]
    m = jnp.max(x, axis=1, keepdims=True)
    lse = m + jnp.log(jnp.sum(jnp.exp(x - m), axis=1, keepdims=True))
    o_ref[...] = x - lse


def _tc_final(g0, g1, d, b2):
    return pl.pallas_call(
        _final_body,
        grid=(N // _BLK,),
        in_specs=[
            pl.BlockSpec((_BLK, D_OUT), lambda i: (i, 0)),
            pl.BlockSpec((_BLK, D_OUT), lambda i: (i, 0)),
            pl.BlockSpec((_BLK, 1), lambda i: (i, 0)),
            pl.BlockSpec((1, D_OUT), lambda i: (0, 0)),
        ],
        out_specs=pl.BlockSpec((_BLK, D_OUT), lambda i: (i, 0)),
        out_shape=jax.ShapeDtypeStruct((N, D_OUT), _f32),
    )(g0, g1, d, b2)


def kernel(features, edge_index, W1, b1, W2, b2):
    src_t = edge_index[0].reshape(NS, CH1, K)
    dst_t = edge_index[1].reshape(NS, CH1, K)
    src_w = edge_index[0].reshape(NW, CH2, K)
    dst_w = edge_index[1].reshape(NW, CH2, K)

    z_half = jnp.zeros((ROWS_PER_TILE, D_HALF), _f32)
    z_deg = jnp.zeros((ROWS_PER_TILE, 16), _f32)
    z_out = jnp.zeros((ROWS_PER_TILE, D_OUT), _f32)
    ones_rows = jnp.ones((K, 16), _f32)

    h1 = _tc_matmul1(features, W1)
    agg1, deg16 = _sc_agg1(h1, src_t, dst_t, z_half, z_deg, ones_rows)

    d = deg16[0, :N, 0:1]
    h2 = _tc_mid(agg1[0, :N], agg1[1, :N], d, b1.reshape(1, D_HID), W2)

    agg2 = _sc_agg2(h2, src_w, dst_w, z_out)
    out = _tc_final(agg2[0, :N], agg2[1, :N], d, b2.reshape(1, D_OUT))
    return out


# speedup vs baseline: 11.3128x; 1.0083x over previous
import functools

import jax
import jax.numpy as jnp
from jax import lax
from jax.experimental import pallas as pl
from jax.experimental.pallas import tpu as pltpu
from jax.experimental.pallas import tpu_sc as plsc

N = 10000
E = 320000
D_IN = 128
D_HID = 128
D_HALF = D_HID // 2
D_OUT = 40

NC = 2
NS = 16
NW = NC * NS
K = 100
E_PER_TILE = E // NS
CH1 = E_PER_TILE // K
E_PER_W = E // NW
CH2 = E_PER_W // K
N_PAD = 10240
ROWS_PER_TILE = N_PAD // NS
NBUF = 4

_f32 = jnp.float32


def _sc_mesh():
    return plsc.VectorSubcoreMesh(core_axis_name="c", subcore_axis_name="s",
                                  num_cores=NC, num_subcores=NS)


def _pipelined_agg(ch, gather_start, gather_wait, scat_start, scat_wait):
    for u in range(NBUF - 1):
        gather_start(u, u)

    def step(j, b):
        gather_wait(j, b)
        scat_start(j, b)
        if not (isinstance(j, int) and j == 0):
            scat_wait(None, (b - 1) % NBUF)
        gather_start(jnp.minimum(j + NBUF - 1, ch - 1), (b + NBUF - 1) % NBUF)

    for j in range(NBUF):
        step(j, j % NBUF)

    def body(t, carry):
        for u in range(NBUF):
            step(t * NBUF + u, u)
        return carry

    lax.fori_loop(1, ch // NBUF, body, 0)

    scat_wait(None, (ch - 1) % NBUF)
    for u in range(NBUF - 1):
        gather_wait(0, u)


@functools.partial(
    pl.kernel,
    out_type=[
        jax.ShapeDtypeStruct((NC, N_PAD, D_HALF), _f32),
        jax.ShapeDtypeStruct((NC, N_PAD, 16), _f32),
    ],
    mesh=_sc_mesh(),
    compiler_params=pltpu.CompilerParams(use_tc_tiling_on_sc=False),
    scratch_types=[
        pltpu.VMEM((CH1, K), jnp.int32),
        pltpu.VMEM((CH1, K), jnp.int32),
        [pltpu.VMEM((K, D_HALF), _f32)] * NBUF,
        pltpu.VMEM((K, 16), _f32),
        pltpu.VMEM_SHARED((N_PAD, D_HALF), _f32),
        pltpu.VMEM_SHARED((N_PAD, 16), _f32),
        [pltpu.SemaphoreType.DMA] * NBUF,
        [pltpu.SemaphoreType.DMA] * NBUF,
    ],
)
def _sc_agg1(h_hbm, src_hbm, dst_hbm, z_feat_hbm, z_deg_hbm, ones_hbm,
             out_hbm, deg_out_hbm,
             sidx, didx, rows, ones_v, acc, dacc, gsem, ssem):
    c = lax.axis_index("c")
    s = lax.axis_index("s")

    pltpu.sync_copy(z_feat_hbm, acc.at[pl.ds(s * ROWS_PER_TILE, ROWS_PER_TILE)])
    pltpu.sync_copy(z_deg_hbm, dacc.at[pl.ds(s * ROWS_PER_TILE, ROWS_PER_TILE)])
    pltpu.sync_copy(src_hbm.at[s], sidx)
    pltpu.sync_copy(dst_hbm.at[s], didx)
    pltpu.sync_copy(ones_hbm, ones_v)
    plsc.subcore_barrier()

    def gather_start(j, b):
        pltpu.async_copy(h_hbm.at[c].at[sidx.at[j]], rows[b], gsem[b])

    def gather_wait(j, b):
        pltpu.make_async_copy(h_hbm.at[c].at[sidx.at[0]], rows[b],
                              gsem[b]).wait()

    def scat_start(j, b):
        pltpu.async_copy(rows[b], acc.at[didx.at[j]], ssem[b], add=True)
        pltpu.async_copy(ones_v, dacc.at[didx.at[j]], ssem[b], add=True)

    def scat_wait(_, b):
        pltpu.make_async_copy(rows[b], acc.at[didx.at[0]], ssem[b]).wait()
        pltpu.make_async_copy(ones_v, dacc.at[didx.at[0]], ssem[b]).wait()

    _pipelined_agg(CH1, gather_start, gather_wait, scat_start, scat_wait)
    plsc.subcore_barrier()

    pltpu.sync_copy(acc.at[pl.ds(s * ROWS_PER_TILE, ROWS_PER_TILE)],
                    out_hbm.at[c, pl.ds(s * ROWS_PER_TILE, ROWS_PER_TILE)])
    pltpu.sync_copy(dacc.at[pl.ds(s * ROWS_PER_TILE, ROWS_PER_TILE)],
                    deg_out_hbm.at[c, pl.ds(s * ROWS_PER_TILE, ROWS_PER_TILE)])


@functools.partial(
    pl.kernel,
    out_type=jax.ShapeDtypeStruct((NC, N_PAD, D_OUT), _f32),
    mesh=_sc_mesh(),
    compiler_params=pltpu.CompilerParams(use_tc_tiling_on_sc=False),
    scratch_types=[
        pltpu.VMEM((CH2, K), jnp.int32),
        pltpu.VMEM((CH2, K), jnp.int32),
        [pltpu.VMEM((K, D_OUT), _f32)] * NBUF,
        pltpu.VMEM_SHARED((N_PAD, D_OUT), _f32),
        [pltpu.SemaphoreType.DMA] * NBUF,
        [pltpu.SemaphoreType.DMA] * NBUF,
    ],
)
def _sc_agg2(h_hbm, src_hbm, dst_hbm, z_feat_hbm,
             out_hbm,
             sidx, didx, rows, acc, gsem, ssem):
    c = lax.axis_index("c")
    s = lax.axis_index("s")
    w = c * NS + s

    pltpu.sync_copy(z_feat_hbm, acc.at[pl.ds(s * ROWS_PER_TILE, ROWS_PER_TILE)])
    pltpu.sync_copy(src_hbm.at[w], sidx)
    pltpu.sync_copy(dst_hbm.at[w], didx)
    plsc.subcore_barrier()

    def gather_start(j, b):
        pltpu.async_copy(h_hbm.at[sidx.at[j]], rows[b], gsem[b])

    def gather_wait(j, b):
        pltpu.make_async_copy(h_hbm.at[sidx.at[0]], rows[b], gsem[b]).wait()

    def scat_start(j, b):
        pltpu.async_copy(rows[b], acc.at[didx.at[j]], ssem[b], add=True)

    def scat_wait(_, b):
        pltpu.make_async_copy(rows[b], acc.at[didx.at[0]], ssem[b]).wait()

    _pipelined_agg(CH2, gather_start, gather_wait, scat_start, scat_wait)
    plsc.subcore_barrier()

    pltpu.sync_copy(acc.at[pl.ds(s * ROWS_PER_TILE, ROWS_PER_TILE)],
                    out_hbm.at[c, pl.ds(s * ROWS_PER_TILE, ROWS_PER_TILE)])


_BLK = 1000


def _mid_body(a0_ref, a1_ref, d_ref, w1_ref, b1_ref, w2_ref, o_ref):
    deg_inv = 1.0 / jnp.maximum(d_ref[...], 1.0)
    ax = jnp.concatenate([a0_ref[...], a1_ref[...]], axis=1)
    h1 = jnp.dot(ax, w1_ref[...], preferred_element_type=_f32)
    x1 = jnp.maximum(h1 * deg_inv + b1_ref[...], 0.0)
    o_ref[...] = jnp.dot(x1, w2_ref[...], preferred_element_type=_f32)


def _tc_mid(a0, a1, d, w1, b1, w2):
    return pl.pallas_call(
        _mid_body,
        grid=(N // _BLK,),
        in_specs=[
            pl.BlockSpec((_BLK, D_HALF), lambda i: (i, 0)),
            pl.BlockSpec((_BLK, D_HALF), lambda i: (i, 0)),
            pl.BlockSpec((_BLK, 1), lambda i: (i, 0)),
            pl.BlockSpec((D_IN, D_HID), lambda i: (0, 0)),
            pl.BlockSpec((1, D_HID), lambda i: (0, 0)),
            pl.BlockSpec((D_HID, D_OUT), lambda i: (0, 0)),
        ],
        out_specs=pl.BlockSpec((_BLK, D_OUT), lambda i: (i, 0)),
        out_shape=jax.ShapeDtypeStruct((N, D_OUT), _f32),
    )(a0, a1, d, w1, b1, w2)


def _final_body(g0_ref, g1_ref, d_ref, b2_ref, o_ref):
    deg_inv = 1.0 / jnp.maximum(d_ref[...], 1.0)
    x = (g0_ref[...] + g1_ref[...]) * deg_inv + b2_ref[...]
    m = jnp.max(x, axis=1, keepdims=True)
    lse = m + jnp.log(jnp.sum(jnp.exp(x - m), axis=1, keepdims=True))
    o_ref[...] = x - lse


def _tc_final(g0, g1, d, b2):
    return pl.pallas_call(
        _final_body,
        grid=(N // _BLK,),
        in_specs=[
            pl.BlockSpec((_BLK, D_OUT), lambda i: (i, 0)),
            pl.BlockSpec((_BLK, D_OUT), lambda i: (i, 0)),
            pl.BlockSpec((_BLK, 1), lambda i: (i, 0)),
            pl.BlockSpec((1, D_OUT), lambda i: (0, 0)),
        ],
        out_specs=pl.BlockSpec((_BLK, D_OUT), lambda i: (i, 0)),
        out_shape=jax.ShapeDtypeStruct((N, D_OUT), _f32),
    )(g0, g1, d, b2)


def kernel(features, edge_index, W1, b1, W2, b2):
    src_t = edge_index[0].reshape(NS, CH1, K)
    dst_t = edge_index[1].reshape(NS, CH1, K)
    src_w = edge_index[0].reshape(NW, CH2, K)
    dst_w = edge_index[1].reshape(NW, CH2, K)

    z_half = jnp.zeros((ROWS_PER_TILE, D_HALF), _f32)
    z_deg = jnp.zeros((ROWS_PER_TILE, 16), _f32)
    z_out = jnp.zeros((ROWS_PER_TILE, D_OUT), _f32)
    ones_rows = jnp.ones((K, 16), _f32)

    xs = features.reshape(N, 2, D_HALF).transpose(1, 0, 2)
    agg1, deg16 = _sc_agg1(xs, src_t, dst_t, z_half, z_deg, ones_rows)

    d = deg16[0, :N, 0:1]
    h2 = _tc_mid(agg1[0, :N], agg1[1, :N], d, W1, b1.reshape(1, D_HID), W2)

    agg2 = _sc_agg2(h2, src_w, dst_w, z_out)
    out = _tc_final(agg2[0, :N], agg2[1, :N], d, b2.reshape(1, D_OUT))
    return out


# speedup vs baseline: 11.9748x; 1.0585x over previous
import functools

import jax
import jax.numpy as jnp
from jax import lax
from jax.experimental import pallas as pl
from jax.experimental.pallas import tpu as pltpu
from jax.experimental.pallas import tpu_sc as plsc

N = 10000
E = 320000
D_IN = 128
D_HID = 128
D_HALF = D_HID // 2
D_OUT = 40

NC = 2
NS = 16
NW = NC * NS
K = 100
E_PER_TILE = E // NS
CH1 = E_PER_TILE // K
E_PER_W = E // NW
CH2 = E_PER_W // K
N_PAD = 10240
ROWS_PER_TILE = N_PAD // NS
NBUF = 4

_f32 = jnp.float32


def _sc_mesh():
    return plsc.VectorSubcoreMesh(core_axis_name="c", subcore_axis_name="s",
                                  num_cores=NC, num_subcores=NS)


def _pipelined_agg(ch, gather_start, gather_wait, scat_start, scat_wait):
    for u in range(NBUF - 1):
        gather_start(u, u)

    def step(j, b):
        gather_wait(j, b)
        scat_start(j, b)
        if not (isinstance(j, int) and j == 0):
            scat_wait(None, (b - 1) % NBUF)
        gather_start(jnp.minimum(j + NBUF - 1, ch - 1), (b + NBUF - 1) % NBUF)

    for j in range(NBUF):
        step(j, j % NBUF)

    def body(t, carry):
        for u in range(NBUF):
            step(t * NBUF + u, u)
        return carry

    lax.fori_loop(1, ch // NBUF, body, 0)

    scat_wait(None, (ch - 1) % NBUF)
    for u in range(NBUF - 1):
        gather_wait(0, u)


@functools.partial(
    pl.kernel,
    out_type=[
        jax.ShapeDtypeStruct((NC, N_PAD, D_HALF), _f32),
        jax.ShapeDtypeStruct((NC, N_PAD, 16), _f32),
    ],
    mesh=_sc_mesh(),
    compiler_params=pltpu.CompilerParams(use_tc_tiling_on_sc=False),
    scratch_types=[
        pltpu.VMEM((CH1, K), jnp.int32),
        pltpu.VMEM((CH1, K), jnp.int32),
        [pltpu.VMEM((K, D_HALF), _f32)] * NBUF,
        pltpu.VMEM((K, 16), _f32),
        pltpu.VMEM_SHARED((N_PAD, D_HALF), _f32),
        pltpu.VMEM_SHARED((N_PAD, 16), _f32),
        [pltpu.SemaphoreType.DMA] * NBUF,
        [pltpu.SemaphoreType.DMA] * NBUF,
    ],
)
def _sc_agg1(h_hbm, src_hbm, dst_hbm, z_feat_hbm, z_deg_hbm, ones_hbm,
             out_hbm, deg_out_hbm,
             sidx, didx, rows, ones_v, acc, dacc, gsem, ssem):
    c = lax.axis_index("c")
    s = lax.axis_index("s")

    pltpu.sync_copy(z_feat_hbm, acc.at[pl.ds(s * ROWS_PER_TILE, ROWS_PER_TILE)])
    pltpu.sync_copy(z_deg_hbm, dacc.at[pl.ds(s * ROWS_PER_TILE, ROWS_PER_TILE)])
    pltpu.sync_copy(src_hbm.at[s], sidx)
    pltpu.sync_copy(dst_hbm.at[s], didx)
    pltpu.sync_copy(ones_hbm, ones_v)
    plsc.subcore_barrier()

    def gather_start(j, b):
        pltpu.async_copy(h_hbm.at[c].at[sidx.at[j]], rows[b], gsem[b])

    def gather_wait(j, b):
        pltpu.make_async_copy(h_hbm.at[c].at[sidx.at[0]], rows[b],
                              gsem[b]).wait()

    def scat_start(j, b):
        pltpu.async_copy(rows[b], acc.at[didx.at[j]], ssem[b], add=True)
        pltpu.async_copy(ones_v, dacc.at[didx.at[j]], ssem[b], add=True)

    def scat_wait(_, b):
        pltpu.make_async_copy(rows[b], acc.at[didx.at[0]], ssem[b]).wait()
        pltpu.make_async_copy(ones_v, dacc.at[didx.at[0]], ssem[b]).wait()

    _pipelined_agg(CH1, gather_start, gather_wait, scat_start, scat_wait)
    plsc.subcore_barrier()

    pltpu.sync_copy(acc.at[pl.ds(s * ROWS_PER_TILE, ROWS_PER_TILE)],
                    out_hbm.at[c, pl.ds(s * ROWS_PER_TILE, ROWS_PER_TILE)])
    pltpu.sync_copy(dacc.at[pl.ds(s * ROWS_PER_TILE, ROWS_PER_TILE)],
                    deg_out_hbm.at[c, pl.ds(s * ROWS_PER_TILE, ROWS_PER_TILE)])


@functools.partial(
    pl.kernel,
    out_type=jax.ShapeDtypeStruct((NC, N_PAD, D_OUT), _f32),
    mesh=_sc_mesh(),
    compiler_params=pltpu.CompilerParams(use_tc_tiling_on_sc=False),
    scratch_types=[
        pltpu.VMEM((CH2, K), jnp.int32),
        pltpu.VMEM((CH2, K), jnp.int32),
        [pltpu.VMEM((K, D_OUT), _f32)] * NBUF,
        pltpu.VMEM_SHARED((N_PAD, D_OUT), _f32),
        [pltpu.SemaphoreType.DMA] * NBUF,
        [pltpu.SemaphoreType.DMA] * NBUF,
    ],
)
def _sc_agg2(h_hbm, src_hbm, dst_hbm, z_feat_hbm,
             out_hbm,
             sidx, didx, rows, acc, gsem, ssem):
    c = lax.axis_index("c")
    s = lax.axis_index("s")
    w = c * NS + s

    pltpu.sync_copy(z_feat_hbm, acc.at[pl.ds(s * ROWS_PER_TILE, ROWS_PER_TILE)])
    pltpu.sync_copy(src_hbm.at[w], sidx)
    pltpu.sync_copy(dst_hbm.at[w], didx)
    plsc.subcore_barrier()

    def gather_start(j, b):
        pltpu.async_copy(h_hbm.at[sidx.at[j]], rows[b], gsem[b])

    def gather_wait(j, b):
        pltpu.make_async_copy(h_hbm.at[sidx.at[0]], rows[b], gsem[b]).wait()

    def scat_start(j, b):
        pltpu.async_copy(rows[b], acc.at[didx.at[j]], ssem[b], add=True)

    def scat_wait(_, b):
        pltpu.make_async_copy(rows[b], acc.at[didx.at[0]], ssem[b]).wait()

    _pipelined_agg(CH2, gather_start, gather_wait, scat_start, scat_wait)
    plsc.subcore_barrier()

    pltpu.sync_copy(acc.at[pl.ds(s * ROWS_PER_TILE, ROWS_PER_TILE)],
                    out_hbm.at[c, pl.ds(s * ROWS_PER_TILE, ROWS_PER_TILE)])


_BLK = 1000


def _mid_body(a_ref, d_ref, w1_ref, b1_ref, w2_ref, o_ref):
    d = d_ref[...][0, :, 0:1]
    deg_inv = 1.0 / jnp.maximum(d, 1.0)
    a = a_ref[...]
    ax = jnp.concatenate([a[0], a[1]], axis=1)
    h1 = jnp.dot(ax, w1_ref[...], preferred_element_type=_f32)
    x1 = jnp.maximum(h1 * deg_inv + b1_ref[...], 0.0)
    o_ref[...] = jnp.dot(x1, w2_ref[...], preferred_element_type=_f32)


def _tc_mid(a, d, w1, b1, w2):
    return pl.pallas_call(
        _mid_body,
        grid=(N // _BLK,),
        in_specs=[
            pl.BlockSpec((2, _BLK, D_HALF), lambda i: (0, i, 0)),
            pl.BlockSpec((1, _BLK, 16), lambda i: (0, i, 0)),
            pl.BlockSpec((D_IN, D_HID), lambda i: (0, 0)),
            pl.BlockSpec((1, D_HID), lambda i: (0, 0)),
            pl.BlockSpec((D_HID, D_OUT), lambda i: (0, 0)),
        ],
        out_specs=pl.BlockSpec((_BLK, D_OUT), lambda i: (i, 0)),
        out_shape=jax.ShapeDtypeStruct((N, D_OUT), _f32),
    )(a, d, w1, b1, w2)


def _final_body(g_ref, d_ref, b2_ref, o_ref):
    d = d_ref[...][0, :, 0:1]
    deg_inv = 1.0 / jnp.maximum(d, 1.0)
    g = g_ref[...]
    x = (g[0] + g[1]) * deg_inv + b2_ref[...]
    m = jnp.max(x, axis=1, keepdims=True)
    lse = m + jnp.log(jnp.sum(jnp.exp(x - m), axis=1, keepdims=True))
    o_ref[...] = x - lse


def _tc_final(g, d, b2):
    return pl.pallas_call(
        _final_body,
        grid=(N // _BLK,),
        in_specs=[
            pl.BlockSpec((2, _BLK, D_OUT), lambda i: (0, i, 0)),
            pl.BlockSpec((1, _BLK, 16), lambda i: (0, i, 0)),
            pl.BlockSpec((1, D_OUT), lambda i: (0, 0)),
        ],
        out_specs=pl.BlockSpec((_BLK, D_OUT), lambda i: (i, 0)),
        out_shape=jax.ShapeDtypeStruct((N, D_OUT), _f32),
    )(g, d, b2)


def kernel(features, edge_index, W1, b1, W2, b2):
    src_t = edge_index[0].reshape(NS, CH1, K)
    dst_t = edge_index[1].reshape(NS, CH1, K)
    src_w = edge_index[0].reshape(NW, CH2, K)
    dst_w = edge_index[1].reshape(NW, CH2, K)

    z_half = jnp.zeros((ROWS_PER_TILE, D_HALF), _f32)
    z_deg = jnp.zeros((ROWS_PER_TILE, 16), _f32)
    z_out = jnp.zeros((ROWS_PER_TILE, D_OUT), _f32)
    ones_rows = jnp.ones((K, 16), _f32)

    xs = features.reshape(N, 2, D_HALF).transpose(1, 0, 2)
    agg1, deg16 = _sc_agg1(xs, src_t, dst_t, z_half, z_deg, ones_rows)

    h2 = _tc_mid(agg1, deg16, W1, b1.reshape(1, D_HID), W2)

    agg2 = _sc_agg2(h2, src_w, dst_w, z_out)
    out = _tc_final(agg2, deg16, b2.reshape(1, D_OUT))
    return out


# speedup vs baseline: 12.9341x; 1.0801x over previous
import functools

import jax
import jax.numpy as jnp
from jax import lax
from jax.experimental import pallas as pl
from jax.experimental.pallas import tpu as pltpu
from jax.experimental.pallas import tpu_sc as plsc

N = 10000
E = 320000
D_IN = 128
D_HID = 128
D_HALF = D_HID // 2
D_OUT = 40

NC = 2
NS = 16
NW = NC * NS
K = 100
E_PER_TILE = E // NS
CH1 = E_PER_TILE // K
E_PER_W = E // NW
CH2 = E_PER_W // K
N_PAD = 10240
ROWS_PER_TILE = N_PAD // NS
NBUF = 4

_f32 = jnp.float32


def _sc_mesh():
    return plsc.VectorSubcoreMesh(core_axis_name="c", subcore_axis_name="s",
                                  num_cores=NC, num_subcores=NS)


def _pipelined_agg(ch, gather_start, gather_wait, scat_start, scat_wait):
    for u in range(NBUF - 1):
        gather_start(u, u)

    def step(j, b):
        gather_wait(j, b)
        scat_start(j, b)
        if not (isinstance(j, int) and j == 0):
            scat_wait(None, (b - 1) % NBUF)
        gather_start(jnp.minimum(j + NBUF - 1, ch - 1), (b + NBUF - 1) % NBUF)

    for j in range(NBUF):
        step(j, j % NBUF)

    def body(t, carry):
        for u in range(NBUF):
            step(t * NBUF + u, u)
        return carry

    lax.fori_loop(1, ch // NBUF, body, 0)

    scat_wait(None, (ch - 1) % NBUF)
    for u in range(NBUF - 1):
        gather_wait(0, u)


@functools.partial(
    pl.kernel,
    out_type=[
        jax.ShapeDtypeStruct((NC, N_PAD, D_HALF), _f32),
        jax.ShapeDtypeStruct((NC, N_PAD, 8), _f32),
    ],
    mesh=_sc_mesh(),
    compiler_params=pltpu.CompilerParams(use_tc_tiling_on_sc=False),
    scratch_types=[
        pltpu.VMEM((CH1, K), jnp.int32),
        pltpu.VMEM((CH1, K), jnp.int32),
        [pltpu.VMEM((K, D_HALF), _f32)] * NBUF,
        pltpu.VMEM((K, 8), _f32),
        pltpu.VMEM_SHARED((N_PAD, D_HALF), _f32),
        pltpu.VMEM_SHARED((N_PAD, 8), _f32),
        [pltpu.SemaphoreType.DMA] * NBUF,
        [pltpu.SemaphoreType.DMA] * NBUF,
    ],
)
def _sc_agg1(h_hbm, src_hbm, dst_hbm, z_feat_hbm, z_deg_hbm, ones_hbm,
             out_hbm, deg_out_hbm,
             sidx, didx, rows, ones_v, acc, dacc, gsem, ssem):
    c = lax.axis_index("c")
    s = lax.axis_index("s")

    pltpu.sync_copy(z_feat_hbm, acc.at[pl.ds(s * ROWS_PER_TILE, ROWS_PER_TILE)])
    pltpu.sync_copy(z_deg_hbm, dacc.at[pl.ds(s * ROWS_PER_TILE, ROWS_PER_TILE)])
    pltpu.sync_copy(src_hbm.at[c, s], sidx)
    pltpu.sync_copy(dst_hbm.at[s], didx)
    pltpu.sync_copy(ones_hbm, ones_v)
    plsc.subcore_barrier()

    def gather_start(j, b):
        pltpu.async_copy(h_hbm.at[sidx.at[j]], rows[b], gsem[b])

    def gather_wait(j, b):
        pltpu.make_async_copy(h_hbm.at[sidx.at[0]], rows[b],
                              gsem[b]).wait()

    def scat_start(j, b):
        pltpu.async_copy(rows[b], acc.at[didx.at[j]], ssem[b], add=True)
        pltpu.async_copy(ones_v, dacc.at[didx.at[j]], ssem[b], add=True)

    def scat_wait(_, b):
        pltpu.make_async_copy(rows[b], acc.at[didx.at[0]], ssem[b]).wait()
        pltpu.make_async_copy(ones_v, dacc.at[didx.at[0]], ssem[b]).wait()

    _pipelined_agg(CH1, gather_start, gather_wait, scat_start, scat_wait)
    plsc.subcore_barrier()

    pltpu.sync_copy(acc.at[pl.ds(s * ROWS_PER_TILE, ROWS_PER_TILE)],
                    out_hbm.at[c, pl.ds(s * ROWS_PER_TILE, ROWS_PER_TILE)])
    pltpu.sync_copy(dacc.at[pl.ds(s * ROWS_PER_TILE, ROWS_PER_TILE)],
                    deg_out_hbm.at[c, pl.ds(s * ROWS_PER_TILE, ROWS_PER_TILE)])


@functools.partial(
    pl.kernel,
    out_type=jax.ShapeDtypeStruct((NC, N_PAD, D_OUT), _f32),
    mesh=_sc_mesh(),
    compiler_params=pltpu.CompilerParams(use_tc_tiling_on_sc=False),
    scratch_types=[
        pltpu.VMEM((CH2, K), jnp.int32),
        pltpu.VMEM((CH2, K), jnp.int32),
        [pltpu.VMEM((K, D_OUT), _f32)] * NBUF,
        pltpu.VMEM_SHARED((N_PAD, D_OUT), _f32),
        [pltpu.SemaphoreType.DMA] * NBUF,
        [pltpu.SemaphoreType.DMA] * NBUF,
    ],
)
def _sc_agg2(h_hbm, src_hbm, dst_hbm, z_feat_hbm,
             out_hbm,
             sidx, didx, rows, acc, gsem, ssem):
    c = lax.axis_index("c")
    s = lax.axis_index("s")

    pltpu.sync_copy(z_feat_hbm, acc.at[pl.ds(s * ROWS_PER_TILE, ROWS_PER_TILE)])
    pltpu.sync_copy(src_hbm.at[s, pl.ds(c * CH2, CH2)], sidx)
    pltpu.sync_copy(dst_hbm.at[s, pl.ds(c * CH2, CH2)], didx)
    plsc.subcore_barrier()

    def gather_start(j, b):
        pltpu.async_copy(h_hbm.at[sidx.at[j]], rows[b], gsem[b])

    def gather_wait(j, b):
        pltpu.make_async_copy(h_hbm.at[sidx.at[0]], rows[b], gsem[b]).wait()

    def scat_start(j, b):
        pltpu.async_copy(rows[b], acc.at[didx.at[j]], ssem[b], add=True)

    def scat_wait(_, b):
        pltpu.make_async_copy(rows[b], acc.at[didx.at[0]], ssem[b]).wait()

    _pipelined_agg(CH2, gather_start, gather_wait, scat_start, scat_wait)
    plsc.subcore_barrier()

    pltpu.sync_copy(acc.at[pl.ds(s * ROWS_PER_TILE, ROWS_PER_TILE)],
                    out_hbm.at[c, pl.ds(s * ROWS_PER_TILE, ROWS_PER_TILE)])


_BLK = 1000


def _mid_body(a_ref, d_ref, w1_ref, b1_ref, w2_ref, o_ref):
    d = d_ref[...][0, :, 0:1]
    deg_inv = 1.0 / jnp.maximum(d, 1.0)
    a = a_ref[...]
    ax = jnp.concatenate([a[0], a[1]], axis=1)
    h1 = jnp.dot(ax, w1_ref[...], preferred_element_type=_f32)
    x1 = jnp.maximum(h1 * deg_inv + b1_ref[...], 0.0)
    o_ref[...] = jnp.dot(x1, w2_ref[...], preferred_element_type=_f32)


def _tc_mid(a, d, w1, b1, w2):
    return pl.pallas_call(
        _mid_body,
        grid=(N // _BLK,),
        in_specs=[
            pl.BlockSpec((2, _BLK, D_HALF), lambda i: (0, i, 0)),
            pl.BlockSpec((1, _BLK, 8), lambda i: (0, i, 0)),
            pl.BlockSpec((D_IN, D_HID), lambda i: (0, 0)),
            pl.BlockSpec((1, D_HID), lambda i: (0, 0)),
            pl.BlockSpec((D_HID, D_OUT), lambda i: (0, 0)),
        ],
        out_specs=pl.BlockSpec((_BLK, D_OUT), lambda i: (i, 0)),
        out_shape=jax.ShapeDtypeStruct((N, D_OUT), _f32),
    )(a, d, w1, b1, w2)


def _final_body(g_ref, d_ref, b2_ref, o_ref):
    d = d_ref[...][0, :, 0:1]
    deg_inv = 1.0 / jnp.maximum(d, 1.0)
    g = g_ref[...]
    x = (g[0] + g[1]) * deg_inv + b2_ref[...]
    m = jnp.max(x, axis=1, keepdims=True)
    lse = m + jnp.log(jnp.sum(jnp.exp(x - m), axis=1, keepdims=True))
    o_ref[...] = x - lse


def _tc_final(g, d, b2):
    return pl.pallas_call(
        _final_body,
        grid=(N // _BLK,),
        in_specs=[
            pl.BlockSpec((2, _BLK, D_OUT), lambda i: (0, i, 0)),
            pl.BlockSpec((1, _BLK, 8), lambda i: (0, i, 0)),
            pl.BlockSpec((1, D_OUT), lambda i: (0, 0)),
        ],
        out_specs=pl.BlockSpec((_BLK, D_OUT), lambda i: (i, 0)),
        out_shape=jax.ShapeDtypeStruct((N, D_OUT), _f32),
    )(g, d, b2)


def kernel(features, edge_index, W1, b1, W2, b2):
    src_t = edge_index[0].reshape(NS, CH1, K)
    dst_t = edge_index[1].reshape(NS, CH1, K)
    src2 = jnp.stack([2 * src_t, 2 * src_t + 1])
    feat2 = features.reshape(2 * N, D_HALF)

    z_half = jnp.zeros((ROWS_PER_TILE, D_HALF), _f32)
    z_deg = jnp.zeros((ROWS_PER_TILE, 8), _f32)
    z_out = jnp.zeros((ROWS_PER_TILE, D_OUT), _f32)
    ones_rows = jnp.ones((K, 8), _f32)

    agg1, deg16 = _sc_agg1(feat2, src2, dst_t, z_half, z_deg, ones_rows)

    h2 = _tc_mid(agg1, deg16, W1, b1.reshape(1, D_HID), W2)

    agg2 = _sc_agg2(h2, src_t, dst_t, z_out)
    out = _tc_final(agg2, deg16, b2.reshape(1, D_OUT))
    return out
